# baseline (speedup 1.0000x reference)
import functools

import jax
import jax.numpy as jnp
from jax import lax
from jax.experimental import pallas as pl
from jax.experimental.pallas import tpu as pltpu
from jax.experimental.pallas import tpu_sc as plsc

_EPS = 1e-5

_NC = 2
_NS = 16
_NW = _NC * _NS


@functools.partial(jax.jit, static_argnames=("chunk",))
def _sc_gather(gidx, tab_flat, chunk=3328):
    n, = gidx.shape
    _, d = tab_flat.shape
    n_per_w = n // _NW
    assert n_per_w * _NW == n and n_per_w % chunk == 0
    n_ch = n_per_w // chunk

    mesh = plsc.VectorSubcoreMesh(
        core_axis_name="c", subcore_axis_name="s",
        num_cores=_NC, num_subcores=_NS)

    @functools.partial(
        pl.kernel,
        out_type=jax.ShapeDtypeStruct((n, d), jnp.bfloat16),
        mesh=mesh,
        scratch_types=[
            pltpu.VMEM((chunk,), jnp.int32),
            pltpu.VMEM((chunk, d), jnp.bfloat16),
            pltpu.SemaphoreType.DMA,
        ],
        compiler_params=pltpu.CompilerParams(use_tc_tiling_on_sc=False),
    )
    def gather(idx_hbm, tab_hbm, out_hbm, idx_v, rows_v, sem):
        wid = lax.axis_index("s") * _NC + lax.axis_index("c")
        base = wid * n_per_w
        for k in range(n_ch):
            off = base + k * chunk
            pltpu.sync_copy(idx_hbm.at[pl.ds(off, chunk)], idx_v)
            pltpu.async_copy(tab_hbm.at[idx_v], rows_v, sem).wait()
            pltpu.sync_copy(rows_v, out_hbm.at[pl.ds(off, chunk)])

    return gather(gidx, tab_flat)


def _tr_body(t_ref, o_ref):
    o_ref[0] = jnp.transpose(t_ref[...], (1, 0)).astype(jnp.bfloat16)


def _mm2_body(x_ref, w_ref, b_ref, h_ref):
    s = pl.program_id(1)

    @pl.when(s == 0)
    def _():
        h_ref[...] = jnp.broadcast_to(b_ref[0], h_ref.shape)

    h_ref[...] += lax.dot_general(
        x_ref[0], w_ref[0], (((1,), (0,)), ((), ())),
        preferred_element_type=jnp.float32)


def _bn_body(h_ref, g_ref, bt_ref, o_ref):
    h = h_ref[...]
    dout2 = h.shape[1]
    s = jnp.sum(h, axis=0, keepdims=True)
    s2 = jnp.sum(h * h, axis=0, keepdims=True)
    cnt = 2.0 * h.shape[0]
    mu = (s[:, :dout2 // 2] + s[:, dout2 // 2:]) / cnt
    ex2 = (s2[:, :dout2 // 2] + s2[:, dout2 // 2:]) / cnt
    var = ex2 - mu * mu
    mu2 = jnp.concatenate([mu, mu], axis=1)
    rs2 = jnp.concatenate([lax.rsqrt(var + _EPS)] * 2, axis=1)
    o_ref[...] = jnp.maximum((h - mu2) * rs2 * g_ref[...] + bt_ref[...], 0.0)


def kernel(features, tables, W, b, gamma, beta):
    bsz, f_num = features.shape
    _, vocab, d = tables.shape
    n = bsz * f_num

    v = features.astype(jnp.int32)
    farange = jnp.arange(f_num, dtype=jnp.int32)
    gidx = ((farange // 4 * vocab)[None, :] + v) * 4 + (farange % 4)[None, :]
    gidx = gidx.reshape(n)

    n_grp = (f_num + 3) // 4
    tab_t = jnp.transpose(tables, (0, 2, 1)).reshape(f_num * d, vocab)
    tab_p = pl.pallas_call(
        _tr_body,
        grid=(n_grp, 8),
        in_specs=[pl.BlockSpec((4 * d, 12800), lambda g, t: (g, t))],
        out_specs=pl.BlockSpec((1, 12800, 4 * d), lambda g, t: (g, t, 0)),
        out_shape=jax.ShapeDtypeStruct((n_grp, vocab, 4 * d), jnp.bfloat16),
    )(tab_t)
    tab_flat = tab_p.reshape(n_grp * vocab * 4, d)

    npair = bsz // 2
    p_iota = jnp.arange(n, dtype=jnp.int32)
    s_p, rem = p_iota // (4 * npair), p_iota % (4 * npair)
    perm = 52 * (rem // 4) + 4 * s_p + rem % 4
    gidx_p = jnp.take(gidx, perm)
    x3 = _sc_gather(gidx_p, tab_flat).reshape(13, npair, 4 * d)

    fan_in = f_num * d
    z = jnp.zeros((fan_in, d), jnp.float32)
    wc3 = jnp.concatenate(
        [jnp.concatenate([W.T, z], axis=1),
         jnp.concatenate([z, W.T], axis=1)],
        axis=0).astype(jnp.bfloat16).reshape(13, 4 * d, 2 * d)
    bc = jnp.concatenate([b, b]).reshape(1, 2 * d)

    blk = 2048
    h2 = pl.pallas_call(
        _mm2_body,
        grid=(npair // blk, 13),
        in_specs=[
            pl.BlockSpec((1, blk, 4 * d), lambda i, s: (s, i, 0)),
            pl.BlockSpec((1, 4 * d, 2 * d), lambda i, s: (s, 0, 0)),
            pl.BlockSpec((1, 2 * d), lambda i, s: (0, 0)),
        ],
        out_specs=pl.BlockSpec((blk, 2 * d), lambda i, s: (i, 0)),
        out_shape=jax.ShapeDtypeStruct((npair, 2 * d), jnp.float32),
    )(x3, wc3, bc)

    g2 = jnp.concatenate([gamma, gamma]).reshape(1, 2 * d)
    bt2 = jnp.concatenate([beta, beta]).reshape(1, 2 * d)
    out2 = pl.pallas_call(
        _bn_body,
        out_shape=jax.ShapeDtypeStruct((bsz // 2, 2 * d), jnp.float32),
    )(h2, g2, bt2)
    return out2.reshape(bsz, d)


# speedup vs baseline: 1.3798x; 1.3798x over previous
import functools

import jax
import jax.numpy as jnp
from jax import lax
from jax.experimental import pallas as pl
from jax.experimental.pallas import tpu as pltpu
from jax.experimental.pallas import tpu_sc as plsc

_EPS = 1e-5

_NC = 2
_NS = 16
_NW = _NC * _NS


def _sc_gather(gidx, tab_flat):
    n, = gidx.shape
    _, d = tab_flat.shape
    chunk = n // _NW

    mesh = plsc.VectorSubcoreMesh(
        core_axis_name="c", subcore_axis_name="s",
        num_cores=_NC, num_subcores=_NS)

    @functools.partial(
        pl.kernel,
        out_type=jax.ShapeDtypeStruct((n, d), jnp.float32),
        mesh=mesh,
        scratch_types=[
            pltpu.VMEM((chunk,), jnp.int32),
            pltpu.VMEM((chunk, d), jnp.float32),
            pltpu.SemaphoreType.DMA,
        ],
        compiler_params=pltpu.CompilerParams(use_tc_tiling_on_sc=False),
    )
    def gather(idx_hbm, tab_hbm, out_hbm, idx_v, rows_v, sem):
        wid = lax.axis_index("s") * _NC + lax.axis_index("c")
        base = wid * chunk
        pltpu.sync_copy(idx_hbm.at[pl.ds(base, chunk)], idx_v)
        pltpu.async_copy(tab_hbm.at[idx_v], rows_v, sem).wait()
        pltpu.sync_copy(rows_v, out_hbm.at[pl.ds(base, chunk)])

    return gather(gidx, tab_flat)


def _tr_body(t_ref, o_ref):
    o_ref[...] = jnp.transpose(t_ref[...], (1, 0))


def _mm_body(w_ref, b_ref, *refs):
    x_refs, h_ref = refs[:-1], refs[-1]
    h = jnp.broadcast_to(b_ref[0], h_ref.shape)
    for g, x_ref in enumerate(x_refs):
        h += lax.dot_general(
            x_ref[...], w_ref[g], (((1,), (0,)), ((), ())),
            preferred_element_type=jnp.float32)
    h_ref[...] = h


def _bn_body(h_ref, g_ref, bt_ref, o_ref):
    h = h_ref[...]
    mu = jnp.mean(h, axis=0, keepdims=True)
    var = jnp.mean((h - mu) ** 2, axis=0, keepdims=True)
    o_ref[...] = jnp.maximum(
        (h - mu) * lax.rsqrt(var + _EPS) * g_ref[...] + bt_ref[...], 0.0)


def kernel(features, tables, W, b, gamma, beta):
    bsz, f_num = features.shape
    _, vocab, d = tables.shape
    n_grp = (f_num + 3) // 4

    tab_t = jnp.transpose(tables, (0, 2, 1)).reshape(f_num * d, vocab)

    feats = features.astype(jnp.int32)
    jarange = jnp.arange(4, dtype=jnp.int32)

    xs = []
    for g in range(n_grp):
        tab_g = pl.pallas_call(
            _tr_body,
            grid=(8,),
            in_specs=[pl.BlockSpec((4 * d, 12800), lambda t, g=g: (g, t))],
            out_specs=pl.BlockSpec((12800, 4 * d), lambda t: (t, 0)),
            out_shape=jax.ShapeDtypeStruct((vocab, 4 * d), jnp.float32),
        )(tab_t)
        nf = min(4, f_num - 4 * g)
        vg = feats[:, 4 * g:4 * g + nf]
        if nf < 4:
            vg = jnp.concatenate(
                [vg, jnp.zeros((bsz, 4 - nf), jnp.int32)], axis=1)
        gidx_g = (vg * 4 + jarange[None, :]).reshape(bsz * 4)
        xs.append(_sc_gather(gidx_g, tab_g.reshape(vocab * 4, d)))

    wg = jnp.concatenate(
        [W.T, jnp.zeros((n_grp * 4 * d - f_num * d, d), jnp.float32)],
        axis=0).reshape(n_grp, 4 * d, d)

    blk = 2048
    h = pl.pallas_call(
        _mm_body,
        grid=(bsz // blk,),
        in_specs=[
            pl.BlockSpec((n_grp, 4 * d, d), lambda i: (0, 0, 0)),
            pl.BlockSpec((1, d), lambda i: (0, 0)),
        ] + [pl.BlockSpec((blk, 4 * d), lambda i: (i, 0))] * n_grp,
        out_specs=pl.BlockSpec((blk, d), lambda i: (i, 0)),
        out_shape=jax.ShapeDtypeStruct((bsz, d), jnp.float32),
    )(wg, b.reshape(1, d), *[x.reshape(bsz, 4 * d) for x in xs])

    out = pl.pallas_call(
        _bn_body,
        out_shape=jax.ShapeDtypeStruct((bsz, d), jnp.float32),
    )(h, gamma.reshape(1, d), beta.reshape(1, d))
    return out


# speedup vs baseline: 2.7522x; 1.9946x over previous
import functools

import jax
import jax.numpy as jnp
from jax import lax
from jax.experimental import pallas as pl
from jax.experimental.pallas import tpu as pltpu
from jax.experimental.pallas import tpu_sc as plsc

_EPS = 1e-5

_NC = 2
_NS = 16
_NW = _NC * _NS


@functools.partial(jax.jit, static_argnames=("chunk", "slab", "grpsz"))
def _sc_gather(gidx, tab_flat, chunk=3328, slab=32768, grpsz=52):
    n, = gidx.shape
    _, d = tab_flat.shape
    n_per_w = n // _NW
    assert n_per_w * _NW == n and n_per_w % chunk == 0
    n_ch = n_per_w // chunk

    mesh = plsc.VectorSubcoreMesh(
        core_axis_name="c", subcore_axis_name="s",
        num_cores=_NC, num_subcores=_NS)

    @functools.partial(
        pl.kernel,
        out_type=jax.ShapeDtypeStruct((n, d), jnp.float32),
        mesh=mesh,
        scratch_types=[
            pltpu.VMEM((chunk,), jnp.int32),
            pltpu.VMEM((chunk,), jnp.int32),
            pltpu.VMEM((chunk, d), jnp.float32),
            pltpu.SemaphoreType.DMA,
        ],
        compiler_params=pltpu.CompilerParams(use_tc_tiling_on_sc=False),
    )
    def gather(idx_hbm, tab_hbm, out_hbm, pos_v, idx_v, rows_v, sem):
        wid = lax.axis_index("s") * _NC + lax.axis_index("c")
        base = wid * n_per_w
        lane = lax.iota(jnp.int32, 16)
        slab_shift = slab.bit_length() - 1
        for k in range(n_ch):
            off = base + k * chunk

            def body(m, off=off):
                pv = (off + m * 16) + lane
                rem = jnp.bitwise_and(pv, slab - 1)
                ii = (grpsz * jnp.right_shift(rem, 2)
                      + jnp.left_shift(jnp.right_shift(pv, slab_shift), 2)
                      + jnp.bitwise_and(pv, 3))
                pos_v[pl.ds(m * 16, 16)] = ii

            lax.fori_loop(0, chunk // 16, lambda m, _: (body(m), 0)[1], 0)
            pltpu.async_copy(idx_hbm.at[pos_v], idx_v, sem).wait()
            pltpu.async_copy(tab_hbm.at[idx_v], rows_v, sem).wait()
            pltpu.sync_copy(rows_v, out_hbm.at[pl.ds(off, chunk)])

    return gather(gidx, tab_flat)


def _tr_body(t_ref, o_ref):
    o_ref[0] = jnp.transpose(t_ref[...], (1, 0))


def _mm2_body(x_ref, w_ref, b_ref, h_ref):
    s = pl.program_id(1)

    @pl.when(s == 0)
    def _():
        h_ref[...] = jnp.broadcast_to(b_ref[0], h_ref.shape)

    h_ref[...] += lax.dot_general(
        x_ref[0], w_ref[0], (((1,), (0,)), ((), ())),
        preferred_element_type=jnp.float32)


def _bn_body(h_ref, g_ref, bt_ref, o_ref):
    h = h_ref[...]
    dout2 = h.shape[1]
    s = jnp.sum(h, axis=0, keepdims=True)
    s2 = jnp.sum(h * h, axis=0, keepdims=True)
    cnt = 2.0 * h.shape[0]
    mu = (s[:, :dout2 // 2] + s[:, dout2 // 2:]) / cnt
    ex2 = (s2[:, :dout2 // 2] + s2[:, dout2 // 2:]) / cnt
    var = ex2 - mu * mu
    mu2 = jnp.concatenate([mu, mu], axis=1)
    rs2 = jnp.concatenate([lax.rsqrt(var + _EPS)] * 2, axis=1)
    o_ref[...] = jnp.maximum((h - mu2) * rs2 * g_ref[...] + bt_ref[...], 0.0)


def kernel(features, tables, W, b, gamma, beta):
    bsz, f_num = features.shape
    _, vocab, d = tables.shape
    n = bsz * f_num

    v = features.astype(jnp.int32)
    farange = jnp.arange(f_num, dtype=jnp.int32)
    gidx = ((farange // 4 * vocab)[None, :] + v) * 4 + (farange % 4)[None, :]
    gidx = gidx.reshape(n)

    n_grp = (f_num + 3) // 4
    tab_t = jnp.transpose(tables, (0, 2, 1)).reshape(f_num * d, vocab)
    tab_p = pl.pallas_call(
        _tr_body,
        grid=(n_grp, 8),
        in_specs=[pl.BlockSpec((4 * d, 12800), lambda g, t: (g, t))],
        out_specs=pl.BlockSpec((1, 12800, 4 * d), lambda g, t: (g, t, 0)),
        out_shape=jax.ShapeDtypeStruct((n_grp, vocab, 4 * d), jnp.float32),
    )(tab_t)
    tab_flat = tab_p.reshape(n_grp * vocab * 4, d)

    npair = bsz // 2
    x3 = _sc_gather(gidx, tab_flat, slab=4 * npair,
                    grpsz=2 * f_num).reshape(13, npair, 4 * d)

    fan_in = f_num * d
    z = jnp.zeros((fan_in, d), jnp.float32)
    wc3 = jnp.concatenate(
        [jnp.concatenate([W.T, z], axis=1),
         jnp.concatenate([z, W.T], axis=1)],
        axis=0).reshape(13, 4 * d, 2 * d)
    bc = jnp.concatenate([b, b]).reshape(1, 2 * d)

    blk = 2048
    h2 = pl.pallas_call(
        _mm2_body,
        grid=(npair // blk, 13),
        in_specs=[
            pl.BlockSpec((1, blk, 4 * d), lambda i, s: (s, i, 0)),
            pl.BlockSpec((1, 4 * d, 2 * d), lambda i, s: (s, 0, 0)),
            pl.BlockSpec((1, 2 * d), lambda i, s: (0, 0)),
        ],
        out_specs=pl.BlockSpec((blk, 2 * d), lambda i, s: (i, 0)),
        out_shape=jax.ShapeDtypeStruct((npair, 2 * d), jnp.float32),
    )(x3, wc3, bc)

    g2 = jnp.concatenate([gamma, gamma]).reshape(1, 2 * d)
    bt2 = jnp.concatenate([beta, beta]).reshape(1, 2 * d)
    out2 = pl.pallas_call(
        _bn_body,
        out_shape=jax.ShapeDtypeStruct((bsz // 2, 2 * d), jnp.float32),
    )(h2, g2, bt2)
    return out2.reshape(bsz, d)


# speedup vs baseline: 2.7801x; 1.0101x over previous
import functools

import jax
import jax.numpy as jnp
from jax import lax
from jax.experimental import pallas as pl
from jax.experimental.pallas import tpu as pltpu
from jax.experimental.pallas import tpu_sc as plsc

_EPS = 1e-5

_NC = 2
_NS = 16
_NW = _NC * _NS


@functools.partial(jax.jit, static_argnames=("chunk", "slab", "grpsz"))
def _sc_gather(gidx, tab_flat, chunk=1664, slab=32768, grpsz=52):
    n, = gidx.shape
    _, d = tab_flat.shape
    n_per_w = n // _NW
    assert n_per_w * _NW == n and n_per_w % chunk == 0
    n_ch = n_per_w // chunk

    mesh = plsc.VectorSubcoreMesh(
        core_axis_name="c", subcore_axis_name="s",
        num_cores=_NC, num_subcores=_NS)

    @functools.partial(
        pl.kernel,
        out_type=jax.ShapeDtypeStruct((n, d), jnp.float32),
        mesh=mesh,
        scratch_types=[
            pltpu.VMEM((chunk,), jnp.int32),
            pltpu.VMEM((chunk,), jnp.int32),
            pltpu.VMEM((2, chunk, d), jnp.float32),
            pltpu.SemaphoreType.DMA,
            pltpu.SemaphoreType.DMA,
            pltpu.SemaphoreType.DMA,
        ],
        compiler_params=pltpu.CompilerParams(use_tc_tiling_on_sc=False),
    )
    def gather(idx_hbm, tab_hbm, out_hbm, pos_v, idx_v, rows_v, sem, wb0, wb1):
        wid = lax.axis_index("s") * _NC + lax.axis_index("c")
        base = wid * n_per_w
        lane = lax.iota(jnp.int32, 16)
        slab_shift = slab.bit_length() - 1
        wbsems = (wb0, wb1)
        wb = [None, None]
        for k in range(n_ch):
            off = base + k * chunk

            def body(m, off=off):
                pv = (off + m * 16) + lane
                rem = jnp.bitwise_and(pv, slab - 1)
                ii = (grpsz * jnp.right_shift(rem, 2)
                      + jnp.left_shift(jnp.right_shift(pv, slab_shift), 2)
                      + jnp.bitwise_and(pv, 3))
                pos_v[pl.ds(m * 16, 16)] = ii

            lax.fori_loop(0, chunk // 16, lambda m, _: (body(m), 0)[1], 0)
            pltpu.async_copy(idx_hbm.at[pos_v], idx_v, sem).wait()
            if wb[k % 2] is not None:
                wb[k % 2].wait()
            pltpu.async_copy(tab_hbm.at[idx_v], rows_v.at[k % 2], sem).wait()
            wb[k % 2] = pltpu.async_copy(
                rows_v.at[k % 2], out_hbm.at[pl.ds(off, chunk)], wbsems[k % 2])
        for w in wb:
            if w is not None:
                w.wait()

    return gather(gidx, tab_flat)


def _tr_body(t_ref, o_ref):
    o_ref[0] = jnp.transpose(t_ref[...], (1, 0))


def _mm2_body(x_ref, w_ref, b_ref, h_ref):
    s = pl.program_id(1)

    @pl.when(s == 0)
    def _():
        h_ref[...] = jnp.broadcast_to(b_ref[0], h_ref.shape)

    h_ref[...] += lax.dot_general(
        x_ref[0], w_ref[0], (((1,), (0,)), ((), ())),
        preferred_element_type=jnp.float32)


def _bn_body(h_ref, g_ref, bt_ref, o_ref):
    h = h_ref[...]
    dout2 = h.shape[1]
    s = jnp.sum(h, axis=0, keepdims=True)
    s2 = jnp.sum(h * h, axis=0, keepdims=True)
    cnt = 2.0 * h.shape[0]
    mu = (s[:, :dout2 // 2] + s[:, dout2 // 2:]) / cnt
    ex2 = (s2[:, :dout2 // 2] + s2[:, dout2 // 2:]) / cnt
    var = ex2 - mu * mu
    mu2 = jnp.concatenate([mu, mu], axis=1)
    rs2 = jnp.concatenate([lax.rsqrt(var + _EPS)] * 2, axis=1)
    o_ref[...] = jnp.maximum((h - mu2) * rs2 * g_ref[...] + bt_ref[...], 0.0)


def kernel(features, tables, W, b, gamma, beta):
    bsz, f_num = features.shape
    _, vocab, d = tables.shape
    n = bsz * f_num

    v = features.astype(jnp.int32)
    farange = jnp.arange(f_num, dtype=jnp.int32)
    gidx = ((farange // 4 * vocab)[None, :] + v) * 4 + (farange % 4)[None, :]
    gidx = gidx.reshape(n)

    n_grp = (f_num + 3) // 4
    tab_t = jnp.transpose(tables, (0, 2, 1)).reshape(f_num * d, vocab)
    tab_p = pl.pallas_call(
        _tr_body,
        grid=(n_grp, 8),
        in_specs=[pl.BlockSpec((4 * d, 12800), lambda g, t: (g, t))],
        out_specs=pl.BlockSpec((1, 12800, 4 * d), lambda g, t: (g, t, 0)),
        out_shape=jax.ShapeDtypeStruct((n_grp, vocab, 4 * d), jnp.float32),
    )(tab_t)
    tab_flat = tab_p.reshape(n_grp * vocab * 4, d)

    npair = bsz // 2
    x3 = _sc_gather(gidx, tab_flat, slab=4 * npair,
                    grpsz=2 * f_num).reshape(13, npair, 4 * d)

    fan_in = f_num * d
    z = jnp.zeros((fan_in, d), jnp.float32)
    wc3 = jnp.concatenate(
        [jnp.concatenate([W.T, z], axis=1),
         jnp.concatenate([z, W.T], axis=1)],
        axis=0).reshape(13, 4 * d, 2 * d)
    bc = jnp.concatenate([b, b]).reshape(1, 2 * d)

    blk = 2048
    h2 = pl.pallas_call(
        _mm2_body,
        grid=(npair // blk, 13),
        in_specs=[
            pl.BlockSpec((1, blk, 4 * d), lambda i, s: (s, i, 0)),
            pl.BlockSpec((1, 4 * d, 2 * d), lambda i, s: (s, 0, 0)),
            pl.BlockSpec((1, 2 * d), lambda i, s: (0, 0)),
        ],
        out_specs=pl.BlockSpec((blk, 2 * d), lambda i, s: (i, 0)),
        out_shape=jax.ShapeDtypeStruct((npair, 2 * d), jnp.float32),
    )(x3, wc3, bc)

    g2 = jnp.concatenate([gamma, gamma]).reshape(1, 2 * d)
    bt2 = jnp.concatenate([beta, beta]).reshape(1, 2 * d)
    out2 = pl.pallas_call(
        _bn_body,
        out_shape=jax.ShapeDtypeStruct((bsz // 2, 2 * d), jnp.float32),
    )(h2, g2, bt2)
    return out2.reshape(bsz, d)
